# baseline (speedup 1.0000x reference)
import jax
import jax.numpy as jnp
from jax import lax
from jax.experimental import pallas as pl
from jax.experimental.pallas import tpu as pltpu
from jax.experimental.pallas import tpu_sc as plsc


def _bias_encoder_sc(B, N, H, FE):
    NP1 = N + 1
    PC = 136
    W = H // 2
    JB = N // 16
    EOFF = 513 * W

    mesh = plsc.VectorSubcoreMesh(core_axis_name="c", subcore_axis_name="s")

    def tec(ab, sp, et, tw, tok, tokr, out1, out2,
            tv, t0, t1, c0, c1, spa, spb, eta, etb, aba, abb, tokv, tokrv,
            sia, sib, st0, st1):
        cid = lax.axis_index("c")
        sid = lax.axis_index("s")
        wid = sid * 2 + cid
        pltpu.sync_copy(tw, tv)
        pltpu.sync_copy(tok, tokv)
        pltpu.sync_copy(tokr, tokrv)
        i16 = lax.iota(jnp.int32, 16)
        zero16 = jnp.zeros((16,), jnp.int32)
        lane0 = i16 == 0

        def start_in(b, i, spx, etx, abx, sem):
            pltpu.async_copy(sp.at[b, i, :], spx, sem)
            pltpu.async_copy(et.at[b, i, :], etx, sem)
            pltpu.async_copy(ab.at[b, i + 1, :], abx, sem)

        def wait_in(b, i, spx, etx, abx, sem):
            pltpu.make_async_copy(sp.at[b, i, :], spx, sem).wait()
            pltpu.make_async_copy(et.at[b, i, :], etx, sem).wait()
            pltpu.make_async_copy(ab.at[b, i + 1, :], abx, sem).wait()

        def start_out(b, orow, tile, colv, sem):
            pltpu.async_copy(tile, out1.at[b, :, orow, :], sem)
            pltpu.async_copy(colv, out2.at[b, orow, :], sem)

        def wait_out(b, orow, tile, colv, sem):
            pltpu.make_async_copy(tile, out1.at[b, :, orow, :], sem).wait()
            pltpu.make_async_copy(colv, out2.at[b, orow, :], sem).wait()

        def emit_main(spx, etx, abx, tile, colv):
            av = abx[pl.ds(0, 16)]
            ab0 = jnp.sum(jnp.where(lane0, av, 0.0)) * 2.0
            plsc.store_scatter(tile, [i16, zero16], tokv[pl.ds(0, 16)] + ab0)
            plsc.store_scatter(tile, [i16 + 16, zero16],
                               tokv[pl.ds(16, 16)] + ab0)

            def do_jb(jb, last):
                ab2 = plsc.load_gather(abx, [i16 + (16 * jb + 1)]) * 2.0
                sp16 = plsc.load_gather(spx, [i16 + 16 * jb])
                spf = sp16 * W
                ef = [plsc.load_gather(etx, [i16 + (f * N + 16 * jb)]) * W
                      + EOFF for f in range(FE)]
                cidx = i16 + (16 * jb + 1)
                if last:
                    tmask = i16 < 15
                    cmask = i16 == 15
                for w in range(W):
                    wv = (i16 + w) & (W - 1)
                    gs = [plsc.bitcast(plsc.load_gather(tv, [base + wv]),
                                       jnp.bfloat16)
                          for base in [spf] + ef]
                    while len(gs) > 1:
                        gs = [a + b for a, b in zip(gs[::2], gs[1::2])] \
                            + gs[-1:] * (len(gs) % 2)
                    si = plsc.bitcast(gs[0], jnp.int32)
                    hE = plsc.bitcast(si << 16, jnp.float32) + ab2
                    hO = plsc.bitcast(si & jnp.int32(-65536), jnp.float32) + ab2
                    rE = wv + wv
                    if last:
                        plsc.store_scatter(tile, [rE, cidx], hE, mask=tmask)
                        plsc.store_scatter(tile, [rE + 1, cidx], hO, mask=tmask)
                        plsc.store_scatter(colv, [rE], hE, mask=cmask)
                        plsc.store_scatter(colv, [rE + 1], hO, mask=cmask)
                    else:
                        plsc.store_scatter(tile, [rE, cidx], hE)
                        plsc.store_scatter(tile, [rE + 1, cidx], hO)

            def jb_body(jj, c):
                do_jb(2 * jj, False)
                do_jb(2 * jj + 1, False)
                return c

            lax.fori_loop(0, JB // 2 - 1, jb_body, 0)
            do_jb(JB - 2, False)
            do_jb(JB - 1, True)

        def emit_row0(abx, tile, colv):
            av = plsc.load_gather(abx, [i16 + (N - 15)])
            abN = jnp.sum(jnp.where(i16 == 15, av, 0.0)) * 2.0
            colv[pl.ds(0, 16)] = tokv[pl.ds(0, 16)] + abN
            colv[pl.ds(16, 16)] = tokv[pl.ds(16, 16)] + abN

            def jb_body(jb, c):
                cidx = i16 + 16 * jb
                ab2 = plsc.load_gather(abx, [cidx]) * 2.0
                for w in range(H):
                    plsc.store_scatter(
                        tile, [jnp.full((16,), w, jnp.int32), cidx],
                        ab2 + tokrv[w, :])
                return c

            lax.fori_loop(0, JB, jb_body, 0)

        for bb in range(2):
            b = wid * 2 + bb
            start_in(b, 0, spa, eta, aba, sia)

            def pair(q, c):
                i0 = 2 * q
                wait_in(b, i0, spa, eta, aba, sia)
                start_in(b, i0 + 1, spb, etb, abb, sib)

                @pl.when(q > 0)
                def _():
                    wait_out(b, i0 + 1, t0, c0, st0)

                emit_main(spa, eta, aba, t0, c0)
                start_out(b, i0 + 1, t0, c0, st0)

                wait_in(b, i0 + 1, spb, etb, abb, sib)

                @pl.when(q < N // 2 - 1)
                def _():
                    start_in(b, i0 + 2, spa, eta, aba, sia)

                @pl.when(q > 0)
                def _():
                    wait_out(b, i0 + 2, t1, c1, st1)

                emit_main(spb, etb, abb, t1, c1)
                start_out(b, i0 + 2, t1, c1, st1)
                return c

            lax.fori_loop(0, N // 2, pair, 0)
            wait_out(b, N - 1, t0, c0, st0)
            wait_out(b, N, t1, c1, st1)
            pltpu.sync_copy(ab.at[b, 0, :], aba)
            emit_row0(aba, t0, c0)
            pltpu.sync_copy(t0, out1.at[b, :, 0, :])
            pltpu.sync_copy(c0, out2.at[b, 0, :])

    return pl.kernel(
        tec,
        out_type=(jax.ShapeDtypeStruct((B, H, PC, N), jnp.float32),
                  jax.ShapeDtypeStruct((B, PC, H), jnp.float32)),
        mesh=mesh,
        compiler_params=pltpu.CompilerParams(use_tc_tiling_on_sc=False,
                                             needs_layout_passes=False),
        scratch_types=[
            pltpu.VMEM((1027 * W,), jnp.int32),
            pltpu.VMEM((H, N), jnp.float32),
            pltpu.VMEM((H, N), jnp.float32),
            pltpu.VMEM((H,), jnp.float32),
            pltpu.VMEM((H,), jnp.float32),
            pltpu.VMEM((N,), jnp.int32),
            pltpu.VMEM((N,), jnp.int32),
            pltpu.VMEM((N * FE,), jnp.int32),
            pltpu.VMEM((N * FE,), jnp.int32),
            pltpu.VMEM((PC,), jnp.float32),
            pltpu.VMEM((PC,), jnp.float32),
            pltpu.VMEM((H,), jnp.float32),
            pltpu.VMEM((H, 16), jnp.float32),
            pltpu.SemaphoreType.DMA,
            pltpu.SemaphoreType.DMA,
            pltpu.SemaphoreType.DMA,
            pltpu.SemaphoreType.DMA,
        ],
    )


def kernel(attn_bias, spatial_pos, attn_edge_type, spatial_W, edge_W, token_W):
    B, NP1, _ = attn_bias.shape
    N = NP1 - 1
    H = spatial_W.shape[1]
    FE = attn_edge_type.shape[-1]
    PC = 136

    tb = jnp.concatenate([spatial_W, edge_W / FE], axis=0).astype(jnp.bfloat16)
    tw = lax.bitcast_convert_type(tb.reshape(-1, H // 2, 2),
                                  jnp.int32).reshape(-1)
    tok = token_W.reshape(H).astype(jnp.float32)
    tokr = jnp.tile(tok.reshape(H, 1), (1, 16))
    sp = spatial_pos.astype(jnp.int32)
    et = attn_edge_type.transpose(0, 1, 3, 2).reshape(B, N, FE * N)
    et = et.astype(jnp.int32)
    abp = jnp.pad(attn_bias, ((0, 0), (0, 0), (0, PC - NP1)))

    run = _bias_encoder_sc(B, N, H, FE)
    out1, out2 = run(abp, sp, et, tw, tok, tokr)
    col = out2[:, :NP1, :].transpose(0, 2, 1)[:, :, :, None]
    return jnp.concatenate([out1[:, :, :NP1, :], col], axis=3)


# speedup vs baseline: 1.3948x; 1.3948x over previous
import jax
import jax.numpy as jnp
from jax import lax
from jax.experimental import pallas as pl
from jax.experimental.pallas import tpu as pltpu
from jax.experimental.pallas import tpu_sc as plsc


def _bias_encoder_sc(B, N, H, FE):
    NP1 = N + 1
    PC = 136
    W = H // 2
    JB = N // 16
    EOFF = 513 * W

    mesh = plsc.VectorSubcoreMesh(core_axis_name="c", subcore_axis_name="s")

    def tec(ab, sp, et, tw, tokr, out,
            tv, t0, t1, spa, spb, eta, etb, aba, abb, tokrv,
            sia, sib, st0, st1):
        cid = lax.axis_index("c")
        sid = lax.axis_index("s")
        wid = sid * 2 + cid
        pltpu.sync_copy(tw, tv)
        pltpu.sync_copy(tokr, tokrv)
        i16 = lax.iota(jnp.int32, 16)

        def start_in(b, i, spx, etx, abx, sem):
            pltpu.async_copy(sp.at[b, i, :], spx, sem)
            pltpu.async_copy(et.at[b, i, :], etx, sem)
            pltpu.async_copy(ab.at[b, i + 1, :], abx, sem)

        def wait_in(b, i, spx, etx, abx, sem):
            pltpu.make_async_copy(sp.at[b, i, :], spx, sem).wait()
            pltpu.make_async_copy(et.at[b, i, :], etx, sem).wait()
            pltpu.make_async_copy(ab.at[b, i + 1, :], abx, sem).wait()

        def emit_main(spx, etx, abx, tile):
            def do_jb(jb):
                ab2 = plsc.load_gather(abx, [i16 + (16 * jb + 1)]) * 2.0
                sp16 = plsc.load_gather(spx, [i16 + 16 * jb])
                spf = sp16 * W
                ef = [plsc.load_gather(etx, [i16 + (f * N + 16 * jb)]) * W
                      + EOFF for f in range(FE)]
                cidx = i16 + 16 * jb
                for w in range(W):
                    wv = (i16 + w) & (W - 1)
                    gs = [plsc.bitcast(plsc.load_gather(tv, [base + wv]),
                                       jnp.bfloat16)
                          for base in [spf] + ef]
                    while len(gs) > 1:
                        gs = [a + b for a, b in zip(gs[::2], gs[1::2])] \
                            + gs[-1:] * (len(gs) % 2)
                    si = plsc.bitcast(gs[0], jnp.int32)
                    hE = plsc.bitcast(si << 16, jnp.float32) + ab2
                    hO = plsc.bitcast(si & jnp.int32(-65536), jnp.float32) + ab2
                    rE = wv + wv
                    plsc.store_scatter(tile, [rE, cidx], hE)
                    plsc.store_scatter(tile, [rE + 1, cidx], hO)

            def jb_body(jj, c):
                do_jb(2 * jj)
                do_jb(2 * jj + 1)
                return c

            lax.fori_loop(0, JB // 2, jb_body, 0)

        def emit_row0(abx, tile):
            def jb_body(jb, c):
                cidx = i16 + 16 * jb
                ab2 = plsc.load_gather(abx, [cidx + 1]) * 2.0
                for w in range(H):
                    plsc.store_scatter(
                        tile, [jnp.full((16,), w, jnp.int32), cidx],
                        ab2 + tokrv[w, :])
                return c

            lax.fori_loop(0, JB, jb_body, 0)

        for bb in range(2):
            b = wid * 2 + bb
            start_in(b, 0, spa, eta, aba, sia)

            def pair(q, c):
                i0 = 2 * q
                wait_in(b, i0, spa, eta, aba, sia)
                start_in(b, i0 + 1, spb, etb, abb, sib)

                @pl.when(q > 0)
                def _():
                    pltpu.make_async_copy(t0, out.at[b, :, i0 + 1, :],
                                          st0).wait()

                emit_main(spa, eta, aba, t0)
                pltpu.async_copy(t0, out.at[b, :, i0 + 1, :], st0)

                wait_in(b, i0 + 1, spb, etb, abb, sib)

                @pl.when(q < N // 2 - 1)
                def _():
                    start_in(b, i0 + 2, spa, eta, aba, sia)

                @pl.when(q > 0)
                def _():
                    pltpu.make_async_copy(t1, out.at[b, :, i0 + 2, :],
                                          st1).wait()

                emit_main(spb, etb, abb, t1)
                pltpu.async_copy(t1, out.at[b, :, i0 + 2, :], st1)
                return c

            lax.fori_loop(0, N // 2, pair, 0)
            pltpu.make_async_copy(t0, out.at[b, :, N - 1, :], st0).wait()
            pltpu.make_async_copy(t1, out.at[b, :, N, :], st1).wait()
            pltpu.sync_copy(ab.at[b, 0, :], aba)
            emit_row0(aba, t0)
            pltpu.sync_copy(t0, out.at[b, :, 0, :])

    return pl.kernel(
        tec,
        out_type=jax.ShapeDtypeStruct((B, H, PC, N), jnp.float32),
        mesh=mesh,
        compiler_params=pltpu.CompilerParams(use_tc_tiling_on_sc=False,
                                             needs_layout_passes=False),
        scratch_types=[
            pltpu.VMEM((1027 * W,), jnp.int32),
            pltpu.VMEM((H, N), jnp.float32),
            pltpu.VMEM((H, N), jnp.float32),
            pltpu.VMEM((N,), jnp.int32),
            pltpu.VMEM((N,), jnp.int32),
            pltpu.VMEM((N * FE,), jnp.int32),
            pltpu.VMEM((N * FE,), jnp.int32),
            pltpu.VMEM((PC,), jnp.float32),
            pltpu.VMEM((PC,), jnp.float32),
            pltpu.VMEM((H, 16), jnp.float32),
            pltpu.SemaphoreType.DMA,
            pltpu.SemaphoreType.DMA,
            pltpu.SemaphoreType.DMA,
            pltpu.SemaphoreType.DMA,
        ],
    )


def kernel(attn_bias, spatial_pos, attn_edge_type, spatial_W, edge_W, token_W):
    B, NP1, _ = attn_bias.shape
    N = NP1 - 1
    H = spatial_W.shape[1]
    FE = attn_edge_type.shape[-1]
    PC = 136

    tb = jnp.concatenate([spatial_W, edge_W / FE], axis=0).astype(jnp.bfloat16)
    tw = lax.bitcast_convert_type(tb.reshape(-1, H // 2, 2),
                                  jnp.int32).reshape(-1)
    tok = token_W.reshape(H).astype(jnp.float32)
    tokr = jnp.tile(tok.reshape(H, 1), (1, 16))
    sp = spatial_pos.astype(jnp.int32)
    et = attn_edge_type.transpose(0, 1, 3, 2).reshape(B, N, FE * N)
    et = et.astype(jnp.int32)
    abp = jnp.pad(attn_bias, ((0, 0), (0, 0), (0, PC - NP1)))

    run = _bias_encoder_sc(B, N, H, FE)
    out1 = run(abp, sp, et, tw, tokr)
    col0 = (2.0 * attn_bias[:, None, :, 0] + tok.reshape(1, H, 1))[..., None]
    return jnp.concatenate([col0, out1[:, :, :NP1, :]], axis=3)
